# baseline (speedup 1.0000x reference)
import functools

import jax
import jax.numpy as jnp
from jax.experimental import pallas as pl
from jax.experimental.pallas import tpu as pltpu
from jax.experimental.pallas import tpu_sc as plsc


_V_BLK = 4096
_NUM_WORKERS = 32


def _sc_gather_pairs(table2, idx):
    n = idx.shape[0]
    e2 = table2.shape[1]
    per_w = n // _NUM_WORKERS

    mesh = plsc.VectorSubcoreMesh(core_axis_name="c", subcore_axis_name="s")

    @functools.partial(
        pl.kernel,
        mesh=mesh,
        out_type=jax.ShapeDtypeStruct((n, e2), table2.dtype),
        scratch_types=[
            pltpu.VMEM((per_w,), jnp.int32),
            pltpu.VMEM((per_w, e2), table2.dtype),
            pltpu.SemaphoreType.DMA,
        ],
    )
    def gather_kernel(tbl_hbm, i_hbm, o_hbm, idx_v, rows_v, sem):
        wid = jax.lax.axis_index("s") * 2 + jax.lax.axis_index("c")
        base = wid * per_w
        pltpu.sync_copy(i_hbm.at[pl.ds(base, per_w)], idx_v)
        pltpu.async_copy(tbl_hbm.at[idx_v], rows_v, sem).wait()
        pltpu.sync_copy(rows_v, o_hbm.at[pl.ds(base, per_w)])

    return gather_kernel(table2, idx)


def _mm_body(par_ref, emb2_ref, wt_ref, b_ref, o_ref):
    half = emb2_ref.shape[1] // 2
    emb = jnp.where(par_ref[...] != 0,
                    emb2_ref[:, half:], emb2_ref[:, :half])
    o_ref[...] = jax.lax.dot_general(
        wt_ref[...], emb,
        dimension_numbers=(((0,), (1,)), ((), ())),
        preferred_element_type=jnp.float32,
    ) + b_ref[...]


def _tc_project_t(parity, emb2, Wt, bc):
    e, vocab = Wt.shape
    batch, e2 = emb2.shape
    num_blocks = pl.cdiv(vocab, _V_BLK)
    return pl.pallas_call(
        _mm_body,
        grid=(num_blocks,),
        in_specs=[
            pl.BlockSpec((batch, 1), lambda i: (0, 0)),
            pl.BlockSpec((batch, e2), lambda i: (0, 0)),
            pl.BlockSpec((e, _V_BLK), lambda i: (0, i)),
            pl.BlockSpec((_V_BLK, 1), lambda i: (i, 0)),
        ],
        out_specs=pl.BlockSpec((_V_BLK, batch), lambda i: (i, 0)),
        out_shape=jax.ShapeDtypeStruct((vocab, batch), jnp.float32),
        compiler_params=pltpu.CompilerParams(
            dimension_semantics=("parallel",)),
    )(parity, emb2, Wt, bc)


def kernel(data, table, W, b):
    data = data.astype(jnp.int32)
    vocab, e = table.shape
    table2 = jnp.concatenate([table[0::2], table[1::2]], axis=1)
    emb2 = _sc_gather_pairs(table2, data >> 1)
    parity = (data & 1).reshape(data.shape[0], 1)
    out_t = _tc_project_t(parity, emb2, jnp.transpose(W),
                          b.reshape(vocab, 1))
    return jnp.transpose(out_t)


# speedup vs baseline: 4.0717x; 4.0717x over previous
import functools

import jax
import jax.numpy as jnp
from jax.experimental import pallas as pl
from jax.experimental.pallas import tpu as pltpu
from jax.experimental.pallas import tpu_sc as plsc


_V_BLK = 4096
_NUM_WORKERS = 32


def _sc_gather_pairs(table2, idx):
    n = idx.shape[0]
    e2 = table2.shape[1]
    per_w = n // _NUM_WORKERS

    mesh = plsc.VectorSubcoreMesh(core_axis_name="c", subcore_axis_name="s")

    @functools.partial(
        pl.kernel,
        mesh=mesh,
        out_type=jax.ShapeDtypeStruct((n, e2), table2.dtype),
        scratch_types=[
            pltpu.VMEM((per_w,), jnp.int32),
            pltpu.VMEM((per_w, e2), table2.dtype),
            pltpu.SemaphoreType.DMA,
        ],
    )
    def gather_kernel(tbl_hbm, i_hbm, o_hbm, idx_v, rows_v, sem):
        wid = jax.lax.axis_index("s") * 2 + jax.lax.axis_index("c")
        base = wid * per_w
        pltpu.sync_copy(i_hbm.at[pl.ds(base, per_w)], idx_v)
        pltpu.async_copy(tbl_hbm.at[idx_v], rows_v, sem).wait()
        pltpu.sync_copy(rows_v, o_hbm.at[pl.ds(base, per_w)])

    return gather_kernel(table2, idx)


def _mm_body(par_ref, emb2_ref, wt_ref, b_ref, o_ref):
    half = emb2_ref.shape[1] // 2
    emb = jnp.where(par_ref[...] != 0,
                    emb2_ref[:, half:], emb2_ref[:, :half])
    o_ref[...] = jax.lax.dot_general(
        wt_ref[...], emb,
        dimension_numbers=(((0,), (1,)), ((), ())),
        preferred_element_type=jnp.float32,
    ) + b_ref[...]


def _tc_project_t(parity, emb2, Wt, bc):
    e, vocab = Wt.shape
    batch, e2 = emb2.shape
    num_blocks = pl.cdiv(vocab, _V_BLK)
    return pl.pallas_call(
        _mm_body,
        grid=(num_blocks,),
        in_specs=[
            pl.BlockSpec((batch, 1), lambda i: (0, 0)),
            pl.BlockSpec((batch, e2), lambda i: (0, 0)),
            pl.BlockSpec((e, _V_BLK), lambda i: (0, i)),
            pl.BlockSpec((_V_BLK, 1), lambda i: (i, 0)),
        ],
        out_specs=pl.BlockSpec((_V_BLK, batch), lambda i: (i, 0)),
        out_shape=jax.ShapeDtypeStruct((vocab, batch), jnp.float32),
        compiler_params=pltpu.CompilerParams(
            dimension_semantics=("parallel",)),
    )(parity, emb2, Wt, bc)


def kernel(data, table, W, b):
    data = data.astype(jnp.int32)
    vocab, e = table.shape
    table2 = table.reshape(vocab // 2, 2 * e)
    emb2 = _sc_gather_pairs(table2, data >> 1)
    parity = (data & 1).reshape(data.shape[0], 1)
    out_t = _tc_project_t(parity, emb2, jnp.transpose(W),
                          b.reshape(vocab, 1))
    return jnp.transpose(out_t)
